# baseline (speedup 1.0000x reference)
import functools

import jax
import jax.numpy as jnp
from jax import lax
from jax.experimental import pallas as pl
from jax.experimental.pallas import tpu as pltpu
from jax.experimental.pallas import tpu_sc as plsc

NC = 2
NS = 16
NW = NC * NS
L = 16

VOCAB = 1000000
D = 64
SEQ = 200
TOTAL_ROWS = 4096 * 200
ROWS_PER_W = TOTAL_ROWS // NW
CHUNK = 200
CHUNKS_PER_W = ROWS_PER_W // CHUNK
IDX_MINOR = 100
IDX_ROWS_PER_CHUNK = CHUNK // IDX_MINOR


def _body(idx_hbm, table_hbm, pos_hbm, out_hbm, pos_v, idx_v, rows_v, sem):
    c = lax.axis_index("c")
    s = lax.axis_index("s")
    wid = s * NC + c

    pltpu.sync_copy(pos_hbm, pos_v)

    def chunk_body(i, carry):
        chunk = wid * CHUNKS_PER_W + i
        pltpu.sync_copy(idx_hbm.at[pl.ds(chunk * IDX_ROWS_PER_CHUNK,
                                         IDX_ROWS_PER_CHUNK)], idx_v)
        cp0 = pltpu.async_copy(table_hbm.at[idx_v.at[0]],
                               rows_v.at[pl.ds(0, IDX_MINOR)], sem)
        cp1 = pltpu.async_copy(table_hbm.at[idx_v.at[1]],
                               rows_v.at[pl.ds(IDX_MINOR, IDX_MINOR)], sem)
        cp0.wait()
        cp1.wait()

        def add_body(r, carry2):
            for dd in range(D // L):
                sl = pl.ds(dd * L, L)
                rows_v[r, sl] = rows_v[r, sl] + pos_v[r, sl]
            return carry2

        lax.fori_loop(0, CHUNK, add_body, 0)
        pltpu.sync_copy(rows_v, out_hbm.at[pl.ds(chunk * CHUNK, CHUNK)])
        return carry

    lax.fori_loop(0, CHUNKS_PER_W, chunk_body, 0)


@jax.jit
def _run(idx_flat, token_table, pos_table):
    mesh = plsc.VectorSubcoreMesh(core_axis_name="c", subcore_axis_name="s",
                                  num_cores=NC, num_subcores=NS)
    return pl.kernel(
        _body,
        out_type=jax.ShapeDtypeStruct((TOTAL_ROWS, D), jnp.float32),
        mesh=mesh,
        scratch_types=[
            pltpu.VMEM((SEQ, D), jnp.float32),
            pltpu.VMEM((IDX_ROWS_PER_CHUNK, IDX_MINOR), jnp.int32),
            pltpu.VMEM((CHUNK, D), jnp.float32),
            pltpu.SemaphoreType.DMA,
        ],
        compiler_params=pltpu.CompilerParams(use_tc_tiling_on_sc=False),
    )(idx_flat, token_table, pos_table)


def kernel(x, token_table, pos_table):
    b, seq = x.shape
    idx_flat = x.reshape(b * seq // IDX_MINOR, IDX_MINOR).astype(jnp.int32)
    out = _run(idx_flat, token_table, pos_table)
    return out.reshape(b, seq, D)


# speedup vs baseline: 1.1495x; 1.1495x over previous
import functools

import jax
import jax.numpy as jnp
from jax import lax
from jax.experimental import pallas as pl
from jax.experimental.pallas import tpu as pltpu
from jax.experimental.pallas import tpu_sc as plsc

NC = 2
NS = 16
NW = NC * NS
L = 16

D = 64
SEQ = 200
TOTAL_ROWS = 4096 * 200
ROWS_PER_W = TOTAL_ROWS // NW
CHUNK = 400
N_CHUNKS = ROWS_PER_W // CHUNK
IDX_MINOR = 100
IPC = CHUNK // IDX_MINOR
NBUF = 3
ROW_UNROLL = 4


def _body(idx_hbm, table_hbm, pos_hbm, out_hbm,
          pos_v, idx_v, rows_v, idx_sem, gat_sem, out_sem):
    c = lax.axis_index("c")
    s = lax.axis_index("s")
    wid = s * NC + c
    chunk0 = wid * N_CHUNKS

    pltpu.sync_copy(pos_hbm, pos_v)

    def start_idx(j, b):
        pltpu.async_copy(idx_hbm.at[pl.ds((chunk0 + j) * IPC, IPC)],
                         idx_v.at[b], idx_sem.at[b])

    def wait_idx(b):
        pltpu.make_async_copy(idx_hbm.at[pl.ds(0, IPC)], idx_v.at[b],
                              idx_sem.at[b]).wait()

    def start_gathers(b):
        for q in range(IPC):
            pltpu.async_copy(table_hbm.at[idx_v.at[b, q]],
                             rows_v.at[b, pl.ds(q * IDX_MINOR, IDX_MINOR)],
                             gat_sem.at[b])

    def wait_gathers(b):
        for q in range(IPC):
            pltpu.make_async_copy(table_hbm.at[pl.ds(0, IDX_MINOR)],
                                  rows_v.at[b, pl.ds(q * IDX_MINOR, IDX_MINOR)],
                                  gat_sem.at[b]).wait()

    def start_scatter(j, b):
        pltpu.async_copy(rows_v.at[b],
                         out_hbm.at[pl.ds((chunk0 + j) * CHUNK, CHUNK)],
                         out_sem.at[b])

    def wait_scatter(b):
        pltpu.make_async_copy(rows_v.at[b], out_hbm.at[pl.ds(0, CHUNK)],
                              out_sem.at[b]).wait()

    def add_pos(b):
        def add_rows(r4, carry):
            r0 = r4 * ROW_UNROLL
            for rr in range(ROW_UNROLL):
                r = r0 + rr
                for dd in range(D // L):
                    sl = pl.ds(dd * L, L)
                    pv = pos_v[r, sl]
                    for rep in range(CHUNK // SEQ):
                        plsc.addupdate(rows_v.at[b, rep * SEQ + r, sl], pv)
            return carry
        lax.fori_loop(0, SEQ // ROW_UNROLL, add_rows, 0, unroll=2)

    start_idx(0, 0)
    start_idx(1, 1)
    wait_idx(0)
    start_gathers(0)

    def step(i, carry):
        b = i % NBUF
        nb = (i + 1) % NBUF

        @pl.when(i + 1 < N_CHUNKS)
        def _():
            wait_idx(nb)

            @pl.when(i + 2 < N_CHUNKS)
            def _():
                start_idx(i + 2, (i + 2) % NBUF)

            @pl.when(i + 1 >= NBUF)
            def _():
                wait_scatter(nb)

            start_gathers(nb)

        wait_gathers(b)
        add_pos(b)
        start_scatter(i, b)
        return carry

    lax.fori_loop(0, N_CHUNKS, step, 0)

    for t in range(NBUF):
        wait_scatter((N_CHUNKS - NBUF + t) % NBUF)


@jax.jit
def _run(idx_flat, token_table, pos_table):
    mesh = plsc.VectorSubcoreMesh(core_axis_name="c", subcore_axis_name="s",
                                  num_cores=NC, num_subcores=NS)
    return pl.kernel(
        _body,
        out_type=jax.ShapeDtypeStruct((TOTAL_ROWS, D), jnp.float32),
        mesh=mesh,
        scratch_types=[
            pltpu.VMEM((SEQ, D), jnp.float32),
            pltpu.VMEM((NBUF, IPC, IDX_MINOR), jnp.int32),
            pltpu.VMEM((NBUF, CHUNK, D), jnp.float32),
            pltpu.SemaphoreType.DMA((NBUF,)),
            pltpu.SemaphoreType.DMA((NBUF,)),
            pltpu.SemaphoreType.DMA((NBUF,)),
        ],
        compiler_params=pltpu.CompilerParams(use_tc_tiling_on_sc=False),
    )(idx_flat, token_table, pos_table)


def kernel(x, token_table, pos_table):
    b, seq = x.shape
    idx_flat = x.reshape(b * seq // IDX_MINOR, IDX_MINOR).astype(jnp.int32)
    out = _run(idx_flat, token_table, pos_table)
    return out.reshape(b, seq, D)
